# baseline (speedup 1.0000x reference)
import functools

import jax
import jax.numpy as jnp
from jax import lax
from jax.experimental import pallas as pl
from jax.experimental.pallas import tpu as pltpu
from jax.experimental.pallas import tpu_sc as plsc

LN_EPS = 1e-12

_NC = 2
_NS = 16
_NW = _NC * _NS

_GATHER_CHUNK = 512
_LN_ROWS = 4096


def _sc_gather_body(table_hbm, idx_hbm, out_hbm,
                    idx_v0, idx_v1, rows_v0, rows_v1,
                    gsem0, gsem1, osem0, osem1):
    n_rows = idx_hbm.shape[0]
    per_w = n_rows // _NW
    wid = lax.axis_index("s") * _NC + lax.axis_index("c")
    base = wid * per_w
    c = _GATHER_CHUNK

    @pl.loop(0, per_w // c, step=2)
    def _(i):
        off0 = base + i * c
        off1 = off0 + c
        pltpu.sync_copy(idx_hbm.at[pl.ds(off0, c)], idx_v0)
        g0 = pltpu.async_copy(table_hbm.at[idx_v0], rows_v0, gsem0)
        pltpu.sync_copy(idx_hbm.at[pl.ds(off1, c)], idx_v1)
        g1 = pltpu.async_copy(table_hbm.at[idx_v1], rows_v1, gsem1)
        g0.wait()
        o0 = pltpu.async_copy(rows_v0, out_hbm.at[pl.ds(off0, c)], osem0)
        g1.wait()
        o1 = pltpu.async_copy(rows_v1, out_hbm.at[pl.ds(off1, c)], osem1)
        o0.wait()
        o1.wait()


def _sc_gather(table, flat_ids):
    n_rows = flat_ids.shape[0]
    d = table.shape[1]
    mesh = plsc.VectorSubcoreMesh(core_axis_name="c", subcore_axis_name="s")
    k = pl.kernel(
        _sc_gather_body,
        out_type=jax.ShapeDtypeStruct((n_rows, d), table.dtype),
        mesh=mesh,
        scratch_types=[
            pltpu.VMEM((_GATHER_CHUNK,), jnp.int32),
            pltpu.VMEM((_GATHER_CHUNK,), jnp.int32),
            pltpu.VMEM((_GATHER_CHUNK, d), table.dtype),
            pltpu.VMEM((_GATHER_CHUNK, d), table.dtype),
            pltpu.SemaphoreType.DMA,
            pltpu.SemaphoreType.DMA,
            pltpu.SemaphoreType.DMA,
            pltpu.SemaphoreType.DMA,
        ],
        compiler_params=pltpu.CompilerParams(use_tc_tiling_on_sc=False),
    )
    return k(table, flat_ids)


def _tc_body(xp_ref, g_ref, wt_ref, w_ref, b_ref, o_ref):
    xp = xp_ref[...]
    pid = g_ref[0]
    w20 = wt_ref[...]

    r2 = xp.shape[0]
    k_iota = lax.broadcasted_iota(jnp.int32, (20, r2), 0)
    oh_t = (pid == k_iota).astype(jnp.float32)
    add = lax.dot_general(
        oh_t, w20,
        dimension_numbers=(((0,), (0,)), ((), ())),
        preferred_element_type=jnp.float32,
    )
    x = xp + add

    lane = lax.broadcasted_iota(jnp.int32, (r2, 128), 1)
    in_a = lane < 64
    zero = jnp.zeros_like(x)
    sum_a = jnp.sum(jnp.where(in_a, x, zero), axis=1, keepdims=True)
    sum_t = jnp.sum(x, axis=1, keepdims=True)
    mu = jnp.where(in_a, sum_a, sum_t - sum_a) * (1.0 / 64.0)
    xc = x - mu
    sq = xc * xc
    sq_a = jnp.sum(jnp.where(in_a, sq, zero), axis=1, keepdims=True)
    sq_t = jnp.sum(sq, axis=1, keepdims=True)
    var = jnp.where(in_a, sq_a, sq_t - sq_a) * (1.0 / 64.0)
    inv = lax.rsqrt(var + LN_EPS)
    o_ref[...] = xc * inv * w_ref[...] + b_ref[...]


def _tc_add_ln(packed, pair_ids3, w20, ln_w2, ln_b2):
    n2 = packed.shape[0]
    r2 = _LN_ROWS // 2
    grid = (n2 // r2,)
    return pl.pallas_call(
        _tc_body,
        grid=grid,
        in_specs=[
            pl.BlockSpec((r2, 128), lambda i: (i, 0)),
            pl.BlockSpec((1, 1, r2), lambda i: (i, 0, 0)),
            pl.BlockSpec((20, 128), lambda i: (0, 0)),
            pl.BlockSpec((1, 128), lambda i: (0, 0)),
            pl.BlockSpec((1, 128), lambda i: (0, 0)),
        ],
        out_specs=pl.BlockSpec((r2, 128), lambda i: (i, 0)),
        out_shape=jax.ShapeDtypeStruct((n2, 128), jnp.float32),
    )(packed, pair_ids3, w20, ln_w2, ln_b2)


def kernel(input_ids, gene_ids, gene_table, word_table, ln_weight, ln_bias):
    b, s = input_ids.shape
    v, d = gene_table.shape
    n_rows = b * s

    table_flat = lax.optimization_barrier(gene_table.reshape(v * d))
    table_lin = table_flat.reshape(v, d)

    flat_ids = input_ids.reshape(n_rows).astype(jnp.int32)
    gathered = _sc_gather(table_lin, flat_ids)
    packed = gathered.reshape(n_rows // 2, 128)

    g_full = jnp.concatenate(
        [jnp.full((b, 1), 4, jnp.int32), gene_ids.astype(jnp.int32)], axis=1
    )
    ga = g_full[:, 0::2]
    gb = g_full[:, 1::2]
    r2 = _LN_ROWS // 2
    pair_ids3 = (ga * 4 + gb).reshape(n_rows // _LN_ROWS, 1, r2)

    wt5 = jnp.concatenate(
        [word_table, jnp.zeros((1, d), word_table.dtype)], axis=0
    )
    a_idx = jnp.arange(20) // 4
    b_idx = jnp.arange(20) % 4
    w20 = jnp.concatenate([wt5[a_idx], word_table[b_idx]], axis=1)

    ln_w2 = jnp.concatenate([ln_weight, ln_weight]).reshape(1, 2 * d)
    ln_b2 = jnp.concatenate([ln_bias, ln_bias]).reshape(1, 2 * d)

    out = _tc_add_ln(packed, pair_ids3, w20, ln_w2, ln_b2)
    return out.reshape(b, s, d)


# speedup vs baseline: 1.0009x; 1.0009x over previous
import functools

import jax
import jax.numpy as jnp
from jax import lax
from jax.experimental import pallas as pl
from jax.experimental.pallas import tpu as pltpu
from jax.experimental.pallas import tpu_sc as plsc

LN_EPS = 1e-12

_NC = 2
_NS = 16
_NW = _NC * _NS

_GATHER_CHUNK = 512
_LN_ROWS = 4096


def _sc_gather_body(table_hbm, idx_hbm, out_hbm,
                    idx_v0, idx_v1, rows_v0, rows_v1,
                    gsem0, gsem1, osem0, osem1):
    n_rows = idx_hbm.shape[0]
    per_w = n_rows // _NW
    wid = lax.axis_index("s") * _NC + lax.axis_index("c")
    base = wid * per_w
    c = _GATHER_CHUNK

    @pl.loop(0, per_w // c, step=2)
    def _(i):
        off0 = base + i * c
        off1 = off0 + c
        pltpu.sync_copy(idx_hbm.at[pl.ds(off0, c)], idx_v0)
        g0 = pltpu.async_copy(table_hbm.at[idx_v0], rows_v0, gsem0)
        pltpu.sync_copy(idx_hbm.at[pl.ds(off1, c)], idx_v1)
        g1 = pltpu.async_copy(table_hbm.at[idx_v1], rows_v1, gsem1)
        g0.wait()
        o0 = pltpu.async_copy(rows_v0, out_hbm.at[pl.ds(off0, c)], osem0)
        g1.wait()
        o1 = pltpu.async_copy(rows_v1, out_hbm.at[pl.ds(off1, c)], osem1)
        o0.wait()
        o1.wait()


def _sc_gather(table, flat_ids):
    n_rows = flat_ids.shape[0]
    d = table.shape[1]
    mesh = plsc.VectorSubcoreMesh(core_axis_name="c", subcore_axis_name="s")
    k = pl.kernel(
        _sc_gather_body,
        out_type=jax.ShapeDtypeStruct((n_rows, d), table.dtype),
        mesh=mesh,
        scratch_types=[
            pltpu.VMEM((_GATHER_CHUNK,), jnp.int32),
            pltpu.VMEM((_GATHER_CHUNK,), jnp.int32),
            pltpu.VMEM((_GATHER_CHUNK, d), table.dtype),
            pltpu.VMEM((_GATHER_CHUNK, d), table.dtype),
            pltpu.SemaphoreType.DMA,
            pltpu.SemaphoreType.DMA,
            pltpu.SemaphoreType.DMA,
            pltpu.SemaphoreType.DMA,
        ],
        compiler_params=pltpu.CompilerParams(use_tc_tiling_on_sc=False),
    )
    return k(table, flat_ids)


def _tc_body(xp_ref, g_ref, wt_ref, w_ref, b_ref, o_ref):
    xp = xp_ref[...]
    pid = g_ref[0]
    w20 = wt_ref[...]

    r2 = xp.shape[0]
    k_iota = lax.broadcasted_iota(jnp.int32, (20, r2), 0)
    oh_t = (pid == k_iota).astype(jnp.float32)
    add = lax.dot_general(
        oh_t, w20,
        dimension_numbers=(((0,), (0,)), ((), ())),
        preferred_element_type=jnp.float32,
    )
    x = xp + add

    lane = lax.broadcasted_iota(jnp.int32, (r2, 128), 1)
    in_a = lane < 64
    zero = jnp.zeros_like(x)
    sum_a = jnp.sum(jnp.where(in_a, x, zero), axis=1, keepdims=True)
    sum_t = jnp.sum(x, axis=1, keepdims=True)
    mu = jnp.where(in_a, sum_a, sum_t - sum_a) * (1.0 / 64.0)
    xc = x - mu
    sq = xc * xc
    sq_a = jnp.sum(jnp.where(in_a, sq, zero), axis=1, keepdims=True)
    sq_t = jnp.sum(sq, axis=1, keepdims=True)
    var = jnp.where(in_a, sq_a, sq_t - sq_a) * (1.0 / 64.0)
    inv = lax.rsqrt(var + LN_EPS)
    o_ref[...] = xc * inv * w_ref[...] + b_ref[...]


def _tc_add_ln(packed, pair_ids3, w20, ln_w2, ln_b2):
    n2 = packed.shape[0]
    r2 = _LN_ROWS // 2
    grid = (n2 // r2,)
    return pl.pallas_call(
        _tc_body,
        grid=grid,
        in_specs=[
            pl.BlockSpec((r2, 128), lambda i: (i, 0)),
            pl.BlockSpec((1, 1, r2), lambda i: (i, 0, 0)),
            pl.BlockSpec((20, 128), lambda i: (0, 0)),
            pl.BlockSpec((1, 128), lambda i: (0, 0)),
            pl.BlockSpec((1, 128), lambda i: (0, 0)),
        ],
        out_specs=pl.BlockSpec((r2, 128), lambda i: (i, 0)),
        out_shape=jax.ShapeDtypeStruct((n2, 128), jnp.float32),
    )(packed, pair_ids3, w20, ln_w2, ln_b2)


def kernel(input_ids, gene_ids, gene_table, word_table, ln_weight, ln_bias):
    b, s = input_ids.shape
    v, d = gene_table.shape
    n_rows = b * s

    table_flat = lax.optimization_barrier(gene_table.reshape(v * d))
    table_lin = table_flat.reshape(v, d)

    flat_ids = input_ids.reshape(n_rows).astype(jnp.int32)
    gathered = _sc_gather(table_lin, flat_ids)
    packed = gathered.reshape(n_rows // 2, 128)

    g_full = jnp.concatenate(
        [jnp.full((b, 1), 4, jnp.int32), gene_ids.astype(jnp.int32)], axis=1
    )
    ga = g_full[:, 0::2]
    gb = g_full[:, 1::2]
    r2 = _LN_ROWS // 2
    pair_ids3 = (ga * 4 + gb).reshape(n_rows // _LN_ROWS, 1, r2)

    wt5 = jnp.concatenate(
        [word_table, jnp.zeros((1, d), word_table.dtype)], axis=0
    )
    a_idx = jnp.arange(20) // 4
    b_idx = jnp.arange(20) % 4
    w20 = jnp.concatenate([wt5[a_idx], word_table[b_idx]], axis=1)

    ln_w2 = jnp.concatenate([ln_weight, ln_weight]).reshape(1, 2 * d)
    ln_b2 = jnp.concatenate([ln_bias, ln_bias]).reshape(1, 2 * d)

    out = _tc_add_ln(packed, pair_ids3, w20, ln_w2, ln_b2)
    out_flat = lax.optimization_barrier(out.reshape(n_rows * d))
    return out_flat.reshape(b, s, d)


# speedup vs baseline: 1.4006x; 1.3993x over previous
import functools

import jax
import jax.numpy as jnp
from jax import lax
from jax.experimental import pallas as pl
from jax.experimental.pallas import tpu as pltpu
from jax.experimental.pallas import tpu_sc as plsc

LN_EPS = 1e-12

_NC = 2
_NS = 16
_NW = _NC * _NS

_GATHER_CHUNK = 512


def _sc_gather_body(table_hbm, idx_hbm, oidx_hbm, out_hbm,
                    idx_v0, idx_v1, oidx_v0, oidx_v1, rows_v0, rows_v1,
                    gsem0, gsem1, osem0, osem1):
    n_rows = idx_hbm.shape[0]
    per_w = n_rows // _NW
    wid = lax.axis_index("s") * _NC + lax.axis_index("c")
    base = wid * per_w
    c = _GATHER_CHUNK

    @pl.loop(0, per_w // c, step=2)
    def _(i):
        off0 = base + i * c
        off1 = off0 + c
        pltpu.sync_copy(idx_hbm.at[pl.ds(off0, c)], idx_v0)
        g0 = pltpu.async_copy(table_hbm.at[idx_v0], rows_v0, gsem0)
        pltpu.sync_copy(oidx_hbm.at[pl.ds(off0, c)], oidx_v0)
        pltpu.sync_copy(idx_hbm.at[pl.ds(off1, c)], idx_v1)
        g1 = pltpu.async_copy(table_hbm.at[idx_v1], rows_v1, gsem1)
        pltpu.sync_copy(oidx_hbm.at[pl.ds(off1, c)], oidx_v1)
        g0.wait()
        o0 = pltpu.async_copy(rows_v0, out_hbm.at[oidx_v0], osem0)
        g1.wait()
        o1 = pltpu.async_copy(rows_v1, out_hbm.at[oidx_v1], osem1)
        o0.wait()
        o1.wait()


def _sc_gather(table, flat_ids, out_idx):
    n_rows = flat_ids.shape[0]
    d = table.shape[1]
    mesh = plsc.VectorSubcoreMesh(core_axis_name="c", subcore_axis_name="s")
    k = pl.kernel(
        _sc_gather_body,
        out_type=jax.ShapeDtypeStruct((n_rows, d), table.dtype),
        mesh=mesh,
        scratch_types=[
            pltpu.VMEM((_GATHER_CHUNK,), jnp.int32),
            pltpu.VMEM((_GATHER_CHUNK,), jnp.int32),
            pltpu.VMEM((_GATHER_CHUNK,), jnp.int32),
            pltpu.VMEM((_GATHER_CHUNK,), jnp.int32),
            pltpu.VMEM((_GATHER_CHUNK, d), table.dtype),
            pltpu.VMEM((_GATHER_CHUNK, d), table.dtype),
            pltpu.SemaphoreType.DMA,
            pltpu.SemaphoreType.DMA,
            pltpu.SemaphoreType.DMA,
            pltpu.SemaphoreType.DMA,
        ],
        compiler_params=pltpu.CompilerParams(use_tc_tiling_on_sc=False),
    )
    return k(table, flat_ids, out_idx)


def _tc_body(xp_ref, g_ref, wt_ref, w_ref, b_ref, o_ref):
    xp = xp_ref[...]
    pid = g_ref[0]
    w25 = wt_ref[...]

    r2 = xp.shape[0]
    k_iota = lax.broadcasted_iota(jnp.int32, (25, r2), 0)
    oh_t = (pid == k_iota).astype(jnp.float32)
    add = lax.dot_general(
        oh_t, w25,
        dimension_numbers=(((0,), (0,)), ((), ())),
        preferred_element_type=jnp.float32,
    )
    x = xp + add

    lane = lax.broadcasted_iota(jnp.int32, (r2, 128), 1)
    in_a = lane < 64
    zero = jnp.zeros_like(x)
    sum_a = jnp.sum(jnp.where(in_a, x, zero), axis=1, keepdims=True)
    sum_t = jnp.sum(x, axis=1, keepdims=True)
    mu = jnp.where(in_a, sum_a, sum_t - sum_a) * (1.0 / 64.0)
    xc = x - mu
    sq = xc * xc
    sq_a = jnp.sum(jnp.where(in_a, sq, zero), axis=1, keepdims=True)
    sq_t = jnp.sum(sq, axis=1, keepdims=True)
    var = jnp.where(in_a, sq_a, sq_t - sq_a) * (1.0 / 64.0)
    inv = lax.rsqrt(var + LN_EPS)
    y = xc * inv * w_ref[...] + b_ref[...]

    yt = y.T
    o_ref[0] = jnp.concatenate([yt[0:64, :], yt[64:128, :]], axis=1)


def _tc_add_ln(packed, pair_ids3, w25, ln_w2, ln_b2, s, b):
    r2 = b // 2
    return pl.pallas_call(
        _tc_body,
        grid=(s,),
        in_specs=[
            pl.BlockSpec((r2, 128), lambda i: (i, 0)),
            pl.BlockSpec((1, 1, r2), lambda i: (i, 0, 0)),
            pl.BlockSpec((25, 128), lambda i: (0, 0)),
            pl.BlockSpec((1, 128), lambda i: (0, 0)),
            pl.BlockSpec((1, 128), lambda i: (0, 0)),
        ],
        out_specs=pl.BlockSpec((1, 64, b), lambda i: (i, 0, 0)),
        out_shape=jax.ShapeDtypeStruct((s, 64, b), jnp.float32),
    )(packed, pair_ids3, w25, ln_w2, ln_b2)


def kernel(input_ids, gene_ids, gene_table, word_table, ln_weight, ln_bias):
    b, s = input_ids.shape
    v, d = gene_table.shape
    n_rows = b * s
    h = b // 2

    flat_ids = input_ids.reshape(n_rows).astype(jnp.int32)
    bb = lax.broadcasted_iota(jnp.int32, (b, s), 0)
    ss = lax.broadcasted_iota(jnp.int32, (b, s), 1)
    out_idx = (ss * b + lax.rem(bb, h) * 2 + bb // h).reshape(n_rows)

    gathered = _sc_gather(gene_table, flat_ids, out_idx)
    packed = gathered.reshape(n_rows // 2, 128)

    g_full = jnp.concatenate(
        [jnp.full((b, 1), 4, jnp.int32), gene_ids.astype(jnp.int32)], axis=1
    )
    pair_ids3 = (g_full[:h] * 5 + g_full[h:]).T.reshape(s, 1, h)

    wt5 = jnp.concatenate(
        [word_table, jnp.zeros((1, d), word_table.dtype)], axis=0
    )
    a_idx = jnp.arange(25) // 5
    b_idx = jnp.arange(25) % 5
    w25 = jnp.concatenate([wt5[a_idx], wt5[b_idx]], axis=1)

    ln_w2 = jnp.concatenate([ln_weight, ln_weight]).reshape(1, 2 * d)
    ln_b2 = jnp.concatenate([ln_bias, ln_bias]).reshape(1, 2 * d)

    out3 = _tc_add_ln(packed, pair_ids3, w25, ln_w2, ln_b2, s, b)
    return jnp.transpose(out3, (2, 0, 1))


# speedup vs baseline: 1.5277x; 1.0907x over previous
import functools

import jax
import jax.numpy as jnp
from jax import lax
from jax.experimental import pallas as pl
from jax.experimental.pallas import tpu as pltpu
from jax.experimental.pallas import tpu_sc as plsc

LN_EPS = 1e-12

_NC = 2
_NS = 16
_NW = _NC * _NS

_GATHER_CHUNK = 512


def _sc_gather_body(table_hbm, idx_hbm, oidx_hbm, out_hbm,
                    idx_v0, idx_v1, oidx_v0, oidx_v1, rows_v0, rows_v1,
                    gsem0, gsem1, osem0, osem1):
    n_rows = idx_hbm.shape[0]
    per_w = n_rows // _NW
    wid = lax.axis_index("s") * _NC + lax.axis_index("c")
    base = wid * per_w
    c = _GATHER_CHUNK

    @pl.loop(0, per_w // c, step=2)
    def _(i):
        off0 = base + i * c
        off1 = off0 + c
        pltpu.sync_copy(idx_hbm.at[pl.ds(off0, c)], idx_v0)
        g0 = pltpu.async_copy(table_hbm.at[idx_v0], rows_v0, gsem0)
        pltpu.sync_copy(oidx_hbm.at[pl.ds(off0, c)], oidx_v0)
        pltpu.sync_copy(idx_hbm.at[pl.ds(off1, c)], idx_v1)
        g1 = pltpu.async_copy(table_hbm.at[idx_v1], rows_v1, gsem1)
        pltpu.sync_copy(oidx_hbm.at[pl.ds(off1, c)], oidx_v1)
        g0.wait()
        o0 = pltpu.async_copy(rows_v0, out_hbm.at[oidx_v0], osem0)
        g1.wait()
        o1 = pltpu.async_copy(rows_v1, out_hbm.at[oidx_v1], osem1)
        o0.wait()
        o1.wait()


def _sc_gather(table, flat_ids, out_idx):
    n_rows = flat_ids.shape[0]
    d = table.shape[1]
    mesh = plsc.VectorSubcoreMesh(core_axis_name="c", subcore_axis_name="s")
    k = pl.kernel(
        _sc_gather_body,
        out_type=jax.ShapeDtypeStruct((n_rows, d), table.dtype),
        mesh=mesh,
        scratch_types=[
            pltpu.VMEM((_GATHER_CHUNK,), jnp.int32),
            pltpu.VMEM((_GATHER_CHUNK,), jnp.int32),
            pltpu.VMEM((_GATHER_CHUNK,), jnp.int32),
            pltpu.VMEM((_GATHER_CHUNK,), jnp.int32),
            pltpu.VMEM((_GATHER_CHUNK, d), table.dtype),
            pltpu.VMEM((_GATHER_CHUNK, d), table.dtype),
            pltpu.SemaphoreType.DMA,
            pltpu.SemaphoreType.DMA,
            pltpu.SemaphoreType.DMA,
            pltpu.SemaphoreType.DMA,
        ],
        compiler_params=pltpu.CompilerParams(use_tc_tiling_on_sc=False),
    )
    return k(table, flat_ids, out_idx)


def _tc_body(xp_ref, g_ref, wt_ref, w_ref, b_ref, o_ref):
    xp = xp_ref[...]
    pid = g_ref[0]
    w25 = wt_ref[...]

    r2 = xp.shape[0]
    k_iota = lax.broadcasted_iota(jnp.int32, (25, r2), 0)
    oh_t = (pid == k_iota).astype(jnp.float32)
    add = lax.dot_general(
        oh_t, w25,
        dimension_numbers=(((0,), (0,)), ((), ())),
        preferred_element_type=jnp.float32,
    )
    x = xp + add

    ri = lax.broadcasted_iota(jnp.int32, (128, 128), 0)
    ci = lax.broadcasted_iota(jnp.int32, (128, 128), 1)
    proj = jnp.where((ri < 64) == (ci < 64), 1.0 / 64.0, 0.0)
    mu = lax.dot_general(
        x, proj,
        dimension_numbers=(((1,), (0,)), ((), ())),
        preferred_element_type=jnp.float32,
    )
    xc = x - mu
    var = lax.dot_general(
        xc * xc, proj,
        dimension_numbers=(((1,), (0,)), ((), ())),
        preferred_element_type=jnp.float32,
    )
    inv = lax.rsqrt(var + LN_EPS)
    y = xc * inv * w_ref[...] + b_ref[...]

    yt = y.T
    o_ref[0] = jnp.concatenate([yt[0:64, :], yt[64:128, :]], axis=1)


def _tc_add_ln(packed, pair_ids3, w25, ln_w2, ln_b2, s, b):
    r2 = b // 2
    return pl.pallas_call(
        _tc_body,
        grid=(s,),
        in_specs=[
            pl.BlockSpec((r2, 128), lambda i: (i, 0)),
            pl.BlockSpec((1, 1, r2), lambda i: (i, 0, 0)),
            pl.BlockSpec((25, 128), lambda i: (0, 0)),
            pl.BlockSpec((1, 128), lambda i: (0, 0)),
            pl.BlockSpec((1, 128), lambda i: (0, 0)),
        ],
        out_specs=pl.BlockSpec((1, 64, b), lambda i: (i, 0, 0)),
        out_shape=jax.ShapeDtypeStruct((s, 64, b), jnp.float32),
    )(packed, pair_ids3, w25, ln_w2, ln_b2)


def kernel(input_ids, gene_ids, gene_table, word_table, ln_weight, ln_bias):
    b, s = input_ids.shape
    v, d = gene_table.shape
    n_rows = b * s
    h = b // 2

    flat_ids = input_ids.reshape(n_rows).astype(jnp.int32)
    bb = lax.broadcasted_iota(jnp.int32, (b, s), 0)
    ss = lax.broadcasted_iota(jnp.int32, (b, s), 1)
    out_idx = (ss * b + lax.rem(bb, h) * 2 + bb // h).reshape(n_rows)

    gathered = _sc_gather(gene_table, flat_ids, out_idx)
    packed = gathered.reshape(n_rows // 2, 128)

    g_full = jnp.concatenate(
        [jnp.full((b, 1), 4, jnp.int32), gene_ids.astype(jnp.int32)], axis=1
    )
    pair_ids3 = (g_full[:h] * 5 + g_full[h:]).T.reshape(s, 1, h)

    wt5 = jnp.concatenate(
        [word_table, jnp.zeros((1, d), word_table.dtype)], axis=0
    )
    a_idx = jnp.arange(25) // 5
    b_idx = jnp.arange(25) % 5
    w25 = jnp.concatenate([wt5[a_idx], wt5[b_idx]], axis=1)

    ln_w2 = jnp.concatenate([ln_weight, ln_weight]).reshape(1, 2 * d)
    ln_b2 = jnp.concatenate([ln_bias, ln_bias]).reshape(1, 2 * d)

    out3 = _tc_add_ln(packed, pair_ids3, w25, ln_w2, ln_b2, s, b)
    return jnp.transpose(out3, (2, 0, 1))
